# Initial kernel scaffold; baseline (speedup 1.0000x reference)
import jax
import jax.numpy as jnp
from jax.experimental import pallas as pl


def kernel(feat, depth, sorts_t, idx2, geom):
    raise NotImplementedError("write your pallas kernel here")


# speedup vs baseline: 1.6363x; 1.6363x over previous
import functools

import jax
import jax.numpy as jnp
from jax import lax
from jax.experimental import pallas as pl
from jax.experimental.pallas import tpu as pltpu

C = 256
HW = 960
NX = 200
NY = 200
NVOX = NX * NY
GROWS = 8
GRID = NX // GROWS


def _tc_body(feat_ref, hwm_ref, depm_ref, out_ref):
    f = feat_ref[...]
    iota = lax.broadcasted_iota(jnp.int32, (HW, NY), 0)
    for r in range(GROWS):
        hw0 = (hwm_ref[0, r, :] - 1.0).astype(jnp.int32)
        hw1 = (hwm_ref[1, r, :] - 1.0).astype(jnp.int32)
        p = (jnp.where(iota == hw0[None, :], 1.0, 0.0)
             + jnp.where(iota == hw1[None, :], 1.0, 0.0))
        mm = jnp.dot(f, p.astype(jnp.bfloat16),
                     preferred_element_type=jnp.float32)
        dep = depm_ref[0, r, :] + depm_ref[1, r, :]
        out_ref[:, r, :] = mm * dep[None, :]


def _tc_call(feat_bf, hwm, depm, interpret=False):
    return pl.pallas_call(
        _tc_body,
        grid=(GRID,),
        in_specs=[
            pl.BlockSpec((C, HW), lambda i: (0, 0)),
            pl.BlockSpec((2, GROWS, NY), lambda i: (0, i, 0)),
            pl.BlockSpec((2, GROWS, NY), lambda i: (0, i, 0)),
        ],
        out_specs=pl.BlockSpec((C, GROWS, NY), lambda i: (0, i, 0)),
        out_shape=jax.ShapeDtypeStruct((C, NX, NY), jnp.float32),
        interpret=interpret,
    )(feat_bf, hwm, depm)


def _build_maps_jnp(depth_flat, sorts_t, idx2, geom):
    p = sorts_t[idx2]
    dep = depth_flat[p]
    hw = (p % HW).astype(jnp.float32) + 1.0
    v = geom[:, 0] * NY + geom[:, 1]
    depm = jnp.zeros((NVOX,), jnp.float32).at[v].set(dep)
    hwm = jnp.zeros((NVOX,), jnp.float32).at[v].set(hw)
    depm2 = jnp.stack([depm, jnp.zeros_like(depm)]).reshape(2, NX, NY)
    hwm2 = jnp.stack([hwm, jnp.zeros_like(hwm)]).reshape(2, NX, NY)
    return depm2, hwm2


@jax.jit
def kernel(feat, depth, sorts_t, idx2, geom):
    feat_bf = feat.reshape(C, HW).astype(jnp.bfloat16)
    depth_flat = depth.reshape(-1)
    depm2, hwm2 = _build_maps_jnp(depth_flat, sorts_t, idx2, geom)
    out = _tc_call(feat_bf, hwm2, depm2)
    return out[None]


# speedup vs baseline: 4.0776x; 2.4919x over previous
import functools

import jax
import jax.numpy as jnp
from jax import lax
from jax.experimental import pallas as pl
from jax.experimental.pallas import tpu as pltpu
from jax.experimental.pallas import tpu_sc as plsc

C = 256
HW = 960
NX = 200
NY = 200
NVOX = NX * NY
GROWS = 8
GRID = NX // GROWS

NCORE = 2
NSUB = 16
NW = NCORE * NSUB
ROWS = 8
CHK = ROWS * 128
NV_PAD = NW * CHK
ACC = 40192
ZCH = ACC // NSUB


def _sc_body(idx2_hbm, geom_hbm, sorts_hbm, depth_hbm, dep_out, hw_out,
             idx2_v, p_v, dep_v, hwp_v, v_v, geom_v, zbuf,
             acc_dep, acc_hw, sem):
    cid = lax.axis_index("c")
    sid = lax.axis_index("s")
    wid = cid * NSUB + sid

    def _fill(i, _):
        zbuf[pl.ds(i * 16, 16)] = jnp.zeros((16,), jnp.float32)
        return 0
    lax.fori_loop(0, ZCH // 16, _fill, 0)
    zbase = pl.multiple_of(sid * ZCH, 8)
    pltpu.sync_copy(zbuf, acc_dep.at[pl.ds(zbase, ZCH)])
    pltpu.sync_copy(zbuf, acc_hw.at[pl.ds(zbase, ZCH)])
    plsc.subcore_barrier()

    rbase = pl.multiple_of(wid * ROWS, 8)
    pltpu.sync_copy(idx2_hbm.at[pl.ds(rbase, ROWS), :], idx2_v)
    gbase = pl.multiple_of(wid * (CHK * 4), 8)
    pltpu.sync_copy(geom_hbm.at[pl.ds(gbase, CHK * 4)], geom_v)
    cps = [pltpu.async_copy(sorts_hbm.at[idx2_v.at[j]], p_v.at[j], sem)
           for j in range(ROWS)]
    for cp in cps:
        cp.wait()
    cps = [pltpu.async_copy(depth_hbm.at[p_v.at[j]], dep_v.at[j], sem)
           for j in range(ROWS)]
    for cp in cps:
        cp.wait()

    iota16 = lax.iota(jnp.int32, 16)
    for j in range(ROWS):
        for k in range(8):
            p16 = p_v[j, pl.ds(k * 16, 16)]
            q16 = (p16.astype(jnp.float32) * (1.0 / HW)).astype(jnp.int32)
            hw16 = p16 - q16 * HW
            hw16 = jnp.where(hw16 < 0, hw16 + HW, hw16)
            hw16 = jnp.where(hw16 >= HW, hw16 - HW, hw16)
            hwp_v[j, pl.ds(k * 16, 16)] = hw16.astype(jnp.float32) + 1.0
            gidx = (iota16 + (j * 128 + k * 16)) * 4
            gx16 = plsc.load_gather(geom_v, [gidx])
            gy16 = plsc.load_gather(geom_v, [gidx + 1])
            v_v[j, pl.ds(k * 16, 16)] = gx16 * NY + gy16

    for j in range(ROWS):
        pltpu.sync_copy(dep_v.at[j], acc_dep.at[v_v.at[j]], add=True)
        pltpu.sync_copy(hwp_v.at[j], acc_hw.at[v_v.at[j]], add=True)
    plsc.subcore_barrier()

    obase = pl.multiple_of(cid * ACC + sid * ZCH, 8)
    pltpu.sync_copy(acc_dep.at[pl.ds(zbase, ZCH)], zbuf)
    pltpu.sync_copy(zbuf, dep_out.at[pl.ds(obase, ZCH)])
    pltpu.sync_copy(acc_hw.at[pl.ds(zbase, ZCH)], zbuf)
    pltpu.sync_copy(zbuf, hw_out.at[pl.ds(obase, ZCH)])


_sc_call = functools.partial(
    pl.kernel,
    out_type=(
        jax.ShapeDtypeStruct((NCORE * ACC,), jnp.float32),
        jax.ShapeDtypeStruct((NCORE * ACC,), jnp.float32),
    ),
    mesh=plsc.VectorSubcoreMesh(core_axis_name="c", subcore_axis_name="s"),
    compiler_params=pltpu.CompilerParams(needs_layout_passes=False),
    scratch_types=[
        pltpu.VMEM((ROWS, 128), jnp.int32),
        pltpu.VMEM((ROWS, 128), jnp.int32),
        pltpu.VMEM((ROWS, 128), jnp.float32),
        pltpu.VMEM((ROWS, 128), jnp.float32),
        pltpu.VMEM((ROWS, 128), jnp.int32),
        pltpu.VMEM((CHK * 4,), jnp.int32),
        pltpu.VMEM((ZCH,), jnp.float32),
        pltpu.VMEM_SHARED((ACC,), jnp.float32),
        pltpu.VMEM_SHARED((ACC,), jnp.float32),
        pltpu.SemaphoreType.DMA,
    ],
)(_sc_body)


def _tc_body(feat_ref, hwm_ref, depm_ref, out_ref):
    f = feat_ref[...]
    iota = lax.broadcasted_iota(jnp.int32, (HW, NY), 0)
    for r in range(GROWS):
        hw0 = (hwm_ref[0, r, :] - 1.0).astype(jnp.int32)
        hw1 = (hwm_ref[1, r, :] - 1.0).astype(jnp.int32)
        p = (jnp.where(iota == hw0[None, :], 1.0, 0.0)
             + jnp.where(iota == hw1[None, :], 1.0, 0.0))
        mm = jnp.dot(f, p.astype(jnp.bfloat16),
                     preferred_element_type=jnp.float32)
        dep = depm_ref[0, r, :] + depm_ref[1, r, :]
        out_ref[:, r, :] = mm * dep[None, :]


def _tc_call(feat_bf, hwm, depm, interpret=False):
    return pl.pallas_call(
        _tc_body,
        grid=(GRID,),
        in_specs=[
            pl.BlockSpec((C, HW), lambda i: (0, 0)),
            pl.BlockSpec((2, GROWS, NY), lambda i: (0, i, 0)),
            pl.BlockSpec((2, GROWS, NY), lambda i: (0, i, 0)),
        ],
        out_specs=pl.BlockSpec((C, GROWS, NY), lambda i: (0, i, 0)),
        out_shape=jax.ShapeDtypeStruct((C, NX, NY), jnp.float32),
        interpret=interpret,
    )(feat_bf, hwm, depm)


@jax.jit
def kernel(feat, depth, sorts_t, idx2, geom):
    feat_bf = feat.reshape(C, HW).astype(jnp.bfloat16)
    depth_flat = depth.reshape(-1)
    nv = idx2.shape[0]
    pad = NV_PAD - nv
    idx2_pad = jnp.concatenate(
        [idx2, jnp.broadcast_to(idx2[-1:], (pad,))]).reshape(NW * ROWS, 128)
    geom_pad = jnp.concatenate(
        [geom,
         jnp.broadcast_to(jnp.array([[NX, 0, 0, 0]], jnp.int32), (pad, 4))]
    ).reshape(-1)
    dep_maps, hw_maps = _sc_call(idx2_pad, geom_pad, sorts_t, depth_flat)
    depm2 = dep_maps.reshape(2, ACC)[:, :NVOX].reshape(2, NX, NY)
    hwm2 = hw_maps.reshape(2, ACC)[:, :NVOX].reshape(2, NX, NY)
    out = _tc_call(feat_bf, hwm2, depm2)
    return out[None]


# speedup vs baseline: 7.0219x; 1.7221x over previous
import functools

import jax
import jax.numpy as jnp
from jax import lax
from jax.experimental import pallas as pl
from jax.experimental.pallas import tpu as pltpu
from jax.experimental.pallas import tpu_sc as plsc

C = 256
HW = 960
NX = 200
NY = 200
NVOX = NX * NY
GROWS = 8
GRID = NX // GROWS

NCORE = 2
NSUB = 16
NW = NCORE * NSUB
WROWS = 7
WIN = WROWS * 128
STRIDE = 768
ACC = 208 * 200
ZCH = ACC // NSUB
ZBUF = 2608


def _make_sc_body(s_last, oob_n):
    def _sc_body(idx2_hbm, geom_hbm, sorts_hbm, depth_hbm, dep_out, hw_out,
                 idx2_v, p_v, dep_v, hwp_v, v_v, geom_v, zbuf,
                 acc_dep, acc_hw, sem):
        cid = lax.axis_index("c")
        sid = lax.axis_index("s")
        wid = cid * NSUB + sid
        iota16 = lax.iota(jnp.int32, 16)

        def _fill(i, _):
            zbuf[pl.ds(i * 16, 16)] = jnp.zeros((16,), jnp.float32)
            return 0
        lax.fori_loop(0, ZBUF // 16, _fill, 0)
        zbase = pl.multiple_of(sid * ZCH, 8)
        pltpu.sync_copy(zbuf.at[pl.ds(0, ZCH)],
                        acc_dep.at[pl.ds(zbase, ZCH)])
        pltpu.sync_copy(zbuf.at[pl.ds(0, ZCH)],
                        acc_hw.at[pl.ds(zbase, ZCH)])

        start = jnp.where(wid == NW - 1, s_last, wid * STRIDE)
        start = pl.multiple_of(start, 8)
        pltpu.sync_copy(idx2_hbm.at[pl.ds(start, WIN)], idx2_v)
        pltpu.sync_copy(geom_hbm.at[pl.ds(start * 4, WIN * 4)], geom_v)
        if oob_n:
            @pl.when(wid == NW - 1)
            def _patch():
                g = idx2_v[pl.ds(WIN - 16, 16)]
                idx2_v[pl.ds(WIN - 16, 16)] = jnp.where(
                    iota16 >= 16 - oob_n, 0, g)
        cps = [pltpu.async_copy(sorts_hbm.at[idx2_v.at[pl.ds(j * 128, 128)]],
                                p_v.at[pl.ds(j * 128, 128)], sem)
               for j in range(WROWS)]
        for cp in cps:
            cp.wait()
        cps = [pltpu.async_copy(depth_hbm.at[p_v.at[pl.ds(j * 128, 128)]],
                                dep_v.at[pl.ds(j * 128, 128)], sem)
               for j in range(WROWS)]
        for cp in cps:
            cp.wait()

        for i in range(WIN // 16):
            p16 = p_v[pl.ds(i * 16, 16)]
            q16 = (p16.astype(jnp.float32) * (1.0 / HW)).astype(jnp.int32)
            hw16 = p16 - q16 * HW
            hw16 = jnp.where(hw16 < 0, hw16 + HW, hw16)
            hw16 = jnp.where(hw16 >= HW, hw16 - HW, hw16)
            hwp_v[pl.ds(i * 16, 16)] = hw16.astype(jnp.float32) + 1.0
            gidx = (iota16 + i * 16) * 4
            gx16 = plsc.load_gather(geom_v, [gidx])
            gy16 = plsc.load_gather(geom_v, [gidx + 1])
            v16 = gx16 * NY + gy16
            if oob_n and i == WIN // 16 - 1:
                v16 = jnp.where(
                    jnp.logical_and(wid == NW - 1, iota16 >= 16 - oob_n),
                    NVOX, v16)
            v_v[i // 8, pl.ds((i % 8) * 16, 16)] = v16

        plsc.subcore_barrier()
        for j in range(WROWS):
            pltpu.sync_copy(dep_v.at[pl.ds(j * 128, 128)],
                            acc_dep.at[v_v.at[j]])
            pltpu.sync_copy(hwp_v.at[pl.ds(j * 128, 128)],
                            acc_hw.at[v_v.at[j]])
        plsc.subcore_barrier()

        obase = pl.multiple_of(cid * ACC + sid * ZCH, 8)
        pltpu.sync_copy(acc_dep.at[pl.ds(zbase, ZCH)],
                        zbuf.at[pl.ds(0, ZCH)])
        pltpu.sync_copy(zbuf.at[pl.ds(0, ZCH)], dep_out.at[pl.ds(obase, ZCH)])
        pltpu.sync_copy(acc_hw.at[pl.ds(zbase, ZCH)],
                        zbuf.at[pl.ds(0, ZCH)])
        pltpu.sync_copy(zbuf.at[pl.ds(0, ZCH)], hw_out.at[pl.ds(obase, ZCH)])

    return _sc_body


@functools.lru_cache(maxsize=None)
def _sc_call(s_last, oob_n):
  return functools.partial(
    pl.kernel,
    out_type=(
        jax.ShapeDtypeStruct((NCORE * ACC,), jnp.float32),
        jax.ShapeDtypeStruct((NCORE * ACC,), jnp.float32),
    ),
    mesh=plsc.VectorSubcoreMesh(core_axis_name="c", subcore_axis_name="s",
                                num_cores=NCORE, num_subcores=NSUB),
    compiler_params=pltpu.CompilerParams(needs_layout_passes=False),
    scratch_types=[
        pltpu.VMEM((WIN,), jnp.int32),
        pltpu.VMEM((WIN,), jnp.int32),
        pltpu.VMEM((WIN,), jnp.float32),
        pltpu.VMEM((WIN,), jnp.float32),
        pltpu.VMEM((WROWS, 128), jnp.int32),
        pltpu.VMEM((WIN * 4,), jnp.int32),
        pltpu.VMEM((ZBUF,), jnp.float32),
        pltpu.VMEM_SHARED((ACC,), jnp.float32),
        pltpu.VMEM_SHARED((ACC,), jnp.float32),
        pltpu.SemaphoreType.DMA,
    ],
  )(_make_sc_body(s_last, oob_n))


def _tc_body(feat_ref, hwm_ref, depm_ref, out_ref):
    f = feat_ref[...]
    iota = lax.broadcasted_iota(jnp.int32, (HW, NY), 0)
    for r in range(GROWS):
        hw0 = hwm_ref[0, r, :].astype(jnp.int32)
        hw1 = hwm_ref[1, r, :].astype(jnp.int32)
        hwc = hw0 + hw1 - 1
        p = jnp.where(iota == hwc[None, :], 1.0, 0.0)
        mm = jnp.dot(f, p.astype(jnp.bfloat16),
                     preferred_element_type=jnp.float32)
        dep = depm_ref[0, r, :] + depm_ref[1, r, :]
        out_ref[:, r, :] = mm * dep[None, :]


def _tc_call(feat_bf, hwm, depm, interpret=False):
    return pl.pallas_call(
        _tc_body,
        grid=(GRID,),
        in_specs=[
            pl.BlockSpec((C, HW), lambda i: (0, 0)),
            pl.BlockSpec((2, GROWS, NY), lambda i: (0, i, 0)),
            pl.BlockSpec((2, GROWS, NY), lambda i: (0, i, 0)),
        ],
        out_specs=pl.BlockSpec((C, GROWS, NY), lambda i: (0, i, 0)),
        out_shape=jax.ShapeDtypeStruct((C, NX, NY), jnp.float32),
        interpret=interpret,
    )(feat_bf, hwm, depm)


@jax.jit
def kernel(feat, depth, sorts_t, idx2, geom):
    feat_bf = feat.reshape(C, HW).astype(jnp.bfloat16)
    depth_flat = depth.reshape(-1)
    nv = idx2.shape[0]
    s_last = max(0, -(-(nv - WIN) // 8) * 8)
    oob_n = s_last + WIN - nv
    dep_maps, hw_maps = _sc_call(s_last, oob_n)(
        idx2, geom.reshape(-1), sorts_t, depth_flat)
    depm2 = dep_maps.reshape(2, ACC // NY, NY)
    hwm2 = hw_maps.reshape(2, ACC // NY, NY)
    out = _tc_call(feat_bf, hwm2, depm2)
    return out[None]


# speedup vs baseline: 7.0290x; 1.0010x over previous
import functools

import jax
import jax.numpy as jnp
from jax import lax
from jax.experimental import pallas as pl
from jax.experimental.pallas import tpu as pltpu
from jax.experimental.pallas import tpu_sc as plsc

C = 256
HW = 960
NX = 200
NY = 200
NVOX = NX * NY
GROWS = 8
GRID = NX // GROWS

NCORE = 2
NSUB = 16
NW = NCORE * NSUB
WROWS = 7
WIN = WROWS * 128
STRIDE = 768
ACC = 208 * 200
ZCH = ACC // NSUB
ZBUF = 2608


def _make_sc_body(s_last, oob_n):
    def _sc_body(idx2_hbm, geom_hbm, sorts_hbm, depth_hbm, dep_out, hw_out,
                 idx2_v, p_v, dep_v, hwp_v, v_v, geom_v, zbuf,
                 acc_dep, acc_hw, sem):
        cid = lax.axis_index("c")
        sid = lax.axis_index("s")
        wid = cid * NSUB + sid
        iota16 = lax.iota(jnp.int32, 16)

        def _fill(i, _):
            zbuf[pl.ds(i * 16, 16)] = jnp.zeros((16,), jnp.float32)
            return 0
        lax.fori_loop(0, ZBUF // 16, _fill, 0)
        zbase = pl.multiple_of(sid * ZCH, 8)
        pltpu.sync_copy(zbuf.at[pl.ds(0, ZCH)],
                        acc_dep.at[pl.ds(zbase, ZCH)])
        pltpu.sync_copy(zbuf.at[pl.ds(0, ZCH)],
                        acc_hw.at[pl.ds(zbase, ZCH)])

        start = jnp.where(wid == NW - 1, s_last, wid * STRIDE)
        start = pl.multiple_of(start, 8)
        pltpu.sync_copy(idx2_hbm.at[pl.ds(start, WIN)], idx2_v)
        pltpu.sync_copy(geom_hbm.at[pl.ds(start * 4, WIN * 4)], geom_v)
        if oob_n:
            @pl.when(wid == NW - 1)
            def _patch():
                g = idx2_v[pl.ds(WIN - 16, 16)]
                idx2_v[pl.ds(WIN - 16, 16)] = jnp.where(
                    iota16 >= 16 - oob_n, 0, g)
        cps = [pltpu.async_copy(sorts_hbm.at[idx2_v.at[pl.ds(j * 128, 128)]],
                                p_v.at[pl.ds(j * 128, 128)], sem)
               for j in range(WROWS)]
        for cp in cps:
            cp.wait()
        cps = [pltpu.async_copy(depth_hbm.at[p_v.at[pl.ds(j * 128, 128)]],
                                dep_v.at[pl.ds(j * 128, 128)], sem)
               for j in range(WROWS)]
        for cp in cps:
            cp.wait()

        for i in range(WIN // 16):
            p16 = p_v[pl.ds(i * 16, 16)]
            q16 = (p16.astype(jnp.float32) * (1.0 / HW)).astype(jnp.int32)
            hw16 = p16 - q16 * HW
            hw16 = jnp.where(hw16 < 0, hw16 + HW, hw16)
            hw16 = jnp.where(hw16 >= HW, hw16 - HW, hw16)
            hwp_v[pl.ds(i * 16, 16)] = hw16.astype(jnp.float32) + 1.0
            gidx = (iota16 + i * 16) * 4
            gx16 = plsc.load_gather(geom_v, [gidx])
            gy16 = plsc.load_gather(geom_v, [gidx + 1])
            v16 = gx16 * NY + gy16
            if oob_n and i == WIN // 16 - 1:
                v16 = jnp.where(
                    jnp.logical_and(wid == NW - 1, iota16 >= 16 - oob_n),
                    NVOX, v16)
            v_v[i // 8, pl.ds((i % 8) * 16, 16)] = v16

        plsc.subcore_barrier()
        for j in range(WROWS):
            pltpu.sync_copy(dep_v.at[pl.ds(j * 128, 128)],
                            acc_dep.at[v_v.at[j]])
            pltpu.sync_copy(hwp_v.at[pl.ds(j * 128, 128)],
                            acc_hw.at[v_v.at[j]])
        plsc.subcore_barrier()

        obase = pl.multiple_of(cid * ACC + sid * ZCH, 8)
        pltpu.sync_copy(acc_dep.at[pl.ds(zbase, ZCH)],
                        zbuf.at[pl.ds(0, ZCH)])
        pltpu.sync_copy(zbuf.at[pl.ds(0, ZCH)], dep_out.at[pl.ds(obase, ZCH)])
        pltpu.sync_copy(acc_hw.at[pl.ds(zbase, ZCH)],
                        zbuf.at[pl.ds(0, ZCH)])
        pltpu.sync_copy(zbuf.at[pl.ds(0, ZCH)], hw_out.at[pl.ds(obase, ZCH)])

    return _sc_body


@functools.lru_cache(maxsize=None)
def _sc_call(s_last, oob_n):
  return functools.partial(
    pl.kernel,
    out_type=(
        jax.ShapeDtypeStruct((NCORE * ACC,), jnp.float32),
        jax.ShapeDtypeStruct((NCORE * ACC,), jnp.float32),
    ),
    mesh=plsc.VectorSubcoreMesh(core_axis_name="c", subcore_axis_name="s",
                                num_cores=NCORE, num_subcores=NSUB),
    compiler_params=pltpu.CompilerParams(needs_layout_passes=False),
    scratch_types=[
        pltpu.VMEM((WIN,), jnp.int32),
        pltpu.VMEM((WIN,), jnp.int32),
        pltpu.VMEM((WIN,), jnp.float32),
        pltpu.VMEM((WIN,), jnp.float32),
        pltpu.VMEM((WROWS, 128), jnp.int32),
        pltpu.VMEM((WIN * 4,), jnp.int32),
        pltpu.VMEM((ZBUF,), jnp.float32),
        pltpu.VMEM_SHARED((ACC,), jnp.float32),
        pltpu.VMEM_SHARED((ACC,), jnp.float32),
        pltpu.SemaphoreType.DMA,
    ],
  )(_make_sc_body(s_last, oob_n))


def _tc_body(feat_ref, hwm_ref, depm_ref, out_ref):
    f = feat_ref[...]
    iota = lax.broadcasted_iota(jnp.int32, (HW, NY), 0)
    for r in range(GROWS):
        hw0 = hwm_ref[0, r, :].astype(jnp.int32)
        hw1 = hwm_ref[1, r, :].astype(jnp.int32)
        occ0 = hw0 > 0
        hwc = jnp.where(occ0, hw0, hw1) - 1
        p = jnp.where(iota == hwc[None, :], 1.0, 0.0)
        mm = jnp.dot(f, p.astype(jnp.bfloat16),
                     preferred_element_type=jnp.float32)
        dep = jnp.where(occ0, depm_ref[0, r, :], depm_ref[1, r, :])
        out_ref[:, r, :] = mm * dep[None, :]


def _tc_call(feat_bf, hwm, depm, interpret=False):
    return pl.pallas_call(
        _tc_body,
        grid=(GRID,),
        in_specs=[
            pl.BlockSpec((C, HW), lambda i: (0, 0)),
            pl.BlockSpec((2, GROWS, NY), lambda i: (0, i, 0)),
            pl.BlockSpec((2, GROWS, NY), lambda i: (0, i, 0)),
        ],
        out_specs=pl.BlockSpec((C, GROWS, NY), lambda i: (0, i, 0)),
        out_shape=jax.ShapeDtypeStruct((C, NX, NY), jnp.float32),
        interpret=interpret,
    )(feat_bf, hwm, depm)


@jax.jit
def kernel(feat, depth, sorts_t, idx2, geom):
    feat_bf = feat.reshape(C, HW).astype(jnp.bfloat16)
    depth_flat = depth.reshape(-1)
    nv = idx2.shape[0]
    s_last = max(0, -(-(nv - WIN) // 8) * 8)
    oob_n = s_last + WIN - nv
    dep_maps, hw_maps = _sc_call(s_last, oob_n)(
        idx2, geom.reshape(-1), sorts_t, depth_flat)
    depm2 = dep_maps.reshape(2, ACC // NY, NY)
    hwm2 = hw_maps.reshape(2, ACC // NY, NY)
    out = _tc_call(feat_bf, hwm2, depm2)
    return out[None]
